# Initial kernel scaffold; baseline (speedup 1.0000x reference)
import jax
import jax.numpy as jnp
from jax.experimental import pallas as pl


def kernel(x, edge_index, W1, b1, W2, b2, W3, b3, W4, b4, Wself, Wneigh, bias):
    raise NotImplementedError("write your pallas kernel here")


# speedup vs baseline: 1.2284x; 1.2284x over previous
import functools

import jax
import jax.numpy as jnp
from jax import lax
from jax.experimental import pallas as pl
from jax.experimental.pallas import tpu as pltpu
from jax.experimental.pallas import tpu_sc as plsc

N = 10000
E = 320000
D = 128
DE = D // 8

NC = 2
NS = 16
NW = NC * NS

CH = 128
NCHT = E // CH
CPW = (NCHT + NW - 1) // NW
CPT = (NCHT + NS - 1) // NS

NPAD = 10240
CZ = 2048


def _leaky(v):
    return jnp.where(v >= 0, v, 0.01 * v)


@functools.cache
def _sc_kernels():
    mesh = plsc.VectorSubcoreMesh(core_axis_name="c", subcore_axis_name="s",
                                  num_cores=NC, num_subcores=NS)

    @functools.partial(
        pl.kernel,
        out_type=(
            jax.ShapeDtypeStruct((E, D), jnp.float32),
            jax.ShapeDtypeStruct((E, D), jnp.float32),
        ),
        mesh=mesh,
        scratch_types=[
            pltpu.VMEM((CH,), jnp.int32),
            pltpu.VMEM((CH, D), jnp.float32),
            pltpu.SemaphoreType.DMA,
        ],
    )
    def _gather_sc(x_hbm, src_hbm, dst_hbm, hu_hbm, hv_hbm, idx_v, rows_v,
                   sem):
        wid = lax.axis_index("s") * NC + lax.axis_index("c")

        def body(t, carry):
            chunk = wid + t * NW

            @pl.when(chunk < NCHT)
            def _():
                base = chunk * CH
                pltpu.sync_copy(src_hbm.at[pl.ds(base, CH)], idx_v)
                pltpu.async_copy(x_hbm.at[idx_v], rows_v, sem).wait()
                pltpu.sync_copy(rows_v, hu_hbm.at[pl.ds(base, CH)])
                pltpu.sync_copy(dst_hbm.at[pl.ds(base, CH)], idx_v)
                pltpu.async_copy(x_hbm.at[idx_v], rows_v, sem).wait()
                pltpu.sync_copy(rows_v, hv_hbm.at[pl.ds(base, CH)])

            return carry

        lax.fori_loop(0, CPW, body, 0)

    out_slice = jax.ShapeDtypeStruct((NPAD, DE), jnp.float32)

    @functools.partial(
        pl.kernel,
        out_type=tuple([out_slice] * 8 + [out_slice, out_slice]),
        mesh=mesh,
        scratch_types=[
            pltpu.VMEM((CH,), jnp.int32),
            pltpu.VMEM((CH, DE), jnp.float32),
            pltpu.VMEM((CH, DE), jnp.float32),
            pltpu.VMEM_SHARED((NPAD, DE), jnp.float32),
        ],
    )
    def _scatter_sc(m0, m1, m2, m3, m4, m5, m6, m7, dst_hbm, zs_hbm,
                    s0, s1, s2, s3, s4, s5, s6, s7, dg0, dg1,
                    idx_v, mrow_v, ones_v, sacc):
        cid = lax.axis_index("c")
        sid = lax.axis_index("s")

        one16 = jnp.ones((16,), jnp.float32)
        for r in range(CH):
            ones_v[r] = one16

        def zero_acc():
            @pl.when(sid == 0)
            def _():
                for q in range(NPAD // CZ):
                    pltpu.sync_copy(zs_hbm, sacc.at[pl.ds(q * CZ, CZ)])

        def copy_out(dsts):
            @pl.when(sid == 0)
            def _():
                for c, dref in enumerate(dsts):
                    @pl.when(cid == c)
                    def _():
                        for q in range(NPAD // CZ):
                            pltpu.sync_copy(sacc.at[pl.ds(q * CZ, CZ)],
                                            dref.at[pl.ds(q * CZ, CZ)])

        m_parts = ((m0, m1), (m2, m3), (m4, m5), (m6, m7))
        s_parts = ((s0, s1), (s2, s3), (s4, s5), (s6, s7))
        for p in range(4):
            zero_acc()
            plsc.subcore_barrier()

            for c in range(NC):
                @pl.when(cid == c)
                def _():
                    m_ref = m_parts[p][c]

                    def body(t, carry):
                        chunk = sid + t * NS

                        @pl.when(chunk < NCHT)
                        def _():
                            base = chunk * CH
                            pltpu.sync_copy(dst_hbm.at[pl.ds(base, CH)],
                                            idx_v)
                            pltpu.sync_copy(m_ref.at[pl.ds(base, CH)],
                                            mrow_v)
                            pltpu.sync_copy(mrow_v, sacc.at[idx_v], add=True)

                        return carry

                    lax.fori_loop(0, CPT, body, 0)

            plsc.subcore_barrier()
            copy_out(s_parts[p])

        zero_acc()
        plsc.subcore_barrier()

        def dbody(t, carry):
            chunk = sid + t * NS

            @pl.when((chunk < NCHT) & ((chunk & 1) == cid))
            def _():
                base = chunk * CH
                pltpu.sync_copy(dst_hbm.at[pl.ds(base, CH)], idx_v)
                pltpu.sync_copy(ones_v, sacc.at[idx_v], add=True)

            return carry

        lax.fori_loop(0, CPT, dbody, 0)
        plsc.subcore_barrier()
        copy_out((dg0, dg1))

    return _gather_sc, _scatter_sc


BE = 1600


def _mlp_body(hu_ref, hv_ref, w1_ref, b1_ref, w2_ref, b2_ref, w3_ref, b3_ref,
              w4_ref, b4_ref, e_ref, *m_refs):
    hu = hu_ref[...]
    d = hu - hv_ref[...]
    dn = (((1,), (1,)), ((), ()))
    h = _leaky(lax.dot_general(d, w1_ref[...], dn,
                               preferred_element_type=jnp.float32) + b1_ref[...])
    h = _leaky(lax.dot_general(h, w2_ref[...], dn,
                               preferred_element_type=jnp.float32) + b2_ref[...])
    h = _leaky(lax.dot_general(h, w3_ref[...], dn,
                               preferred_element_type=jnp.float32) + b3_ref[...])
    z = jnp.sum(h * w4_ref[...], axis=1, keepdims=True) + b4_ref[...]
    e = 1.0 / (1.0 + jnp.exp(-z))
    e_ref[...] = e
    m = e * hu
    for i, m_ref in enumerate(m_refs):
        m_ref[...] = m[:, i * DE:(i + 1) * DE]


def _mlp_tc(hu, hv, W1, b1, W2, b2, W3, b3, W4, b4):
    full = lambda s: pl.BlockSpec(s, lambda i: (0, 0))
    espec = pl.BlockSpec((BE, DE), lambda i: (i, 0))
    return pl.pallas_call(
        _mlp_body,
        grid=(E // BE,),
        in_specs=[
            pl.BlockSpec((BE, D), lambda i: (i, 0)),
            pl.BlockSpec((BE, D), lambda i: (i, 0)),
            full((256, D)), full((1, 256)),
            full((128, 256)), full((1, 128)),
            full((64, 128)), full((1, 64)),
            full((1, 64)), full((1, 1)),
        ],
        out_specs=[pl.BlockSpec((BE, 1), lambda i: (i, 0))] + [espec] * 8,
        out_shape=[jax.ShapeDtypeStruct((E, 1), jnp.float32)]
        + [jax.ShapeDtypeStruct((E, DE), jnp.float32)] * 8,
    )(hu, hv, W1, b1.reshape(1, 256), W2, b2.reshape(1, 128),
      W3, b3.reshape(1, 64), W4, b4.reshape(1, 1))


BN = 2000


def _final_body(x_ref, s0, s1, s2, s3, s4, s5, s6, s7, dg0, dg1, ws_ref,
                w0, w1, w2, w3, w4, w5, w6, w7, b_ref, a_ref):
    deg = dg0[:, 0:1] + dg1[:, 0:1]
    inv = 1.0 / jnp.maximum(deg, 1.0)
    dn = (((1,), (1,)), ((), ()))
    rst = lax.dot_general(x_ref[...], ws_ref[...], dn,
                          preferred_element_type=jnp.float32) + b_ref[...]
    for s_ref, w_ref in zip((s0, s1, s2, s3, s4, s5, s6, s7),
                            (w0, w1, w2, w3, w4, w5, w6, w7)):
        rst = rst + lax.dot_general(s_ref[...] * inv, w_ref[...], dn,
                                    preferred_element_type=jnp.float32)
    a = _leaky(rst)
    nrm = jnp.sqrt(jnp.sum(a * a, axis=1, keepdims=True))
    a_ref[...] = a / jnp.maximum(nrm, 1e-12)


def _final_tc(x, s_parts, dg0, dg1, Wself, Wneigh, bias):
    row = pl.BlockSpec((BN, D), lambda i: (i, 0))
    erow = pl.BlockSpec((BN, DE), lambda i: (i, 0))
    full = lambda s: pl.BlockSpec(s, lambda i: (0, 0))
    wspec = full((D, DE))
    w_parts = [Wneigh[:, i * DE:(i + 1) * DE] for i in range(8)]
    return pl.pallas_call(
        _final_body,
        grid=(N // BN,),
        in_specs=[row] + [erow] * 8 + [erow, erow]
        + [full((D, D))] + [wspec] * 8 + [full((1, D))],
        out_specs=row,
        out_shape=jax.ShapeDtypeStruct((N, D), jnp.float32),
    )(x, *s_parts, dg0, dg1, Wself, *w_parts, bias.reshape(1, D))


def kernel(x, edge_index, W1, b1, W2, b2, W3, b3, W4, b4, Wself, Wneigh, bias):
    src = edge_index[0]
    dst = edge_index[1]
    _gather_sc, _scatter_sc = _sc_kernels()
    hu, hv = _gather_sc(x, src, dst)
    e, *m_parts = _mlp_tc(hu, hv, W1, b1, W2, b2, W3, b3, W4, b4)
    zs = jnp.zeros((CZ, DE), jnp.float32)
    outs = _scatter_sc(*m_parts, dst, zs)
    s_parts = [o[:N] for o in outs[:8]]
    dg0, dg1 = outs[8][:N], outs[9][:N]
    A = _final_tc(x, s_parts, dg0, dg1, Wself, Wneigh, bias)
    return (A, e)


# speedup vs baseline: 1.2639x; 1.0289x over previous
import functools

import jax
import jax.numpy as jnp
from jax import lax
from jax.experimental import pallas as pl
from jax.experimental.pallas import tpu as pltpu
from jax.experimental.pallas import tpu_sc as plsc

N = 10000
E = 320000
D = 128
DE = D // 8

NC = 2
NS = 16
NW = NC * NS

CH = 128
NCHT = E // CH
CPW = (NCHT + NW - 1) // NW
CPT = (NCHT + NS - 1) // NS

NPAD = 10240
CZ = 2048


def _leaky(v):
    return jnp.where(v >= 0, v, 0.01 * v)


@functools.cache
def _sc_kernels():
    mesh = plsc.VectorSubcoreMesh(core_axis_name="c", subcore_axis_name="s",
                                  num_cores=NC, num_subcores=NS)

    @functools.partial(
        pl.kernel,
        out_type=(
            jax.ShapeDtypeStruct((E, D), jnp.float32),
            jax.ShapeDtypeStruct((E, D), jnp.float32),
        ),
        mesh=mesh,
        scratch_types=[
            pltpu.VMEM((CH,), jnp.int32),
            pltpu.VMEM((CH, D), jnp.float32),
            pltpu.SemaphoreType.DMA,
        ],
    )
    def _gather_sc(x_hbm, src_hbm, dst_hbm, hu_hbm, hv_hbm, idx_v, rows_v,
                   sem):
        wid = lax.axis_index("s") * NC + lax.axis_index("c")

        def body(t, carry):
            chunk = wid + t * NW

            @pl.when(chunk < NCHT)
            def _():
                base = chunk * CH
                pltpu.sync_copy(src_hbm.at[pl.ds(base, CH)], idx_v)
                pltpu.async_copy(x_hbm.at[idx_v], rows_v, sem).wait()
                pltpu.sync_copy(rows_v, hu_hbm.at[pl.ds(base, CH)])
                pltpu.sync_copy(dst_hbm.at[pl.ds(base, CH)], idx_v)
                pltpu.async_copy(x_hbm.at[idx_v], rows_v, sem).wait()
                pltpu.sync_copy(rows_v, hv_hbm.at[pl.ds(base, CH)])

            return carry

        lax.fori_loop(0, CPW, body, 0)

    out_slice = jax.ShapeDtypeStruct((NPAD, DE), jnp.float32)

    @functools.partial(
        pl.kernel,
        out_type=tuple([out_slice] * 8 + [out_slice, out_slice]),
        mesh=mesh,
        scratch_types=[
            pltpu.VMEM((CH,), jnp.int32),
            pltpu.VMEM((CH, DE), jnp.float32),
            pltpu.VMEM((CH, DE), jnp.float32),
            pltpu.VMEM_SHARED((NPAD, DE), jnp.float32),
        ],
    )
    def _scatter_sc(m0, m1, m2, m3, m4, m5, m6, m7, dst_hbm, zs_hbm,
                    s0, s1, s2, s3, s4, s5, s6, s7, dg0, dg1,
                    idx_v, mrow_v, ones_v, sacc):
        cid = lax.axis_index("c")
        sid = lax.axis_index("s")

        one16 = jnp.ones((16,), jnp.float32)
        for r in range(CH):
            for j in range(DE // 16):
                ones_v[r, pl.ds(j * 16, 16)] = one16

        def zero_acc():
            @pl.when(sid == 0)
            def _():
                for q in range(NPAD // CZ):
                    pltpu.sync_copy(zs_hbm, sacc.at[pl.ds(q * CZ, CZ)])

        def copy_out(dsts):
            @pl.when(sid == 0)
            def _():
                for c, dref in enumerate(dsts):
                    @pl.when(cid == c)
                    def _():
                        for q in range(NPAD // CZ):
                            pltpu.sync_copy(sacc.at[pl.ds(q * CZ, CZ)],
                                            dref.at[pl.ds(q * CZ, CZ)])

        m_parts = ((m0, m1), (m2, m3), (m4, m5), (m6, m7))
        s_parts = ((s0, s1), (s2, s3), (s4, s5), (s6, s7))
        for p in range(4):
            zero_acc()
            plsc.subcore_barrier()

            for c in range(NC):
                @pl.when(cid == c)
                def _():
                    m_ref = m_parts[p][c]

                    def body(t, carry):
                        chunk = sid + t * NS

                        @pl.when(chunk < NCHT)
                        def _():
                            base = chunk * CH
                            pltpu.sync_copy(dst_hbm.at[pl.ds(base, CH)],
                                            idx_v)
                            pltpu.sync_copy(m_ref.at[pl.ds(base, CH)],
                                            mrow_v)
                            pltpu.sync_copy(mrow_v, sacc.at[idx_v], add=True)

                        return carry

                    lax.fori_loop(0, CPT, body, 0)

            plsc.subcore_barrier()
            copy_out(s_parts[p])

        zero_acc()
        plsc.subcore_barrier()

        def dbody(t, carry):
            chunk = sid + t * NS

            @pl.when((chunk < NCHT) & ((chunk & 1) == cid))
            def _():
                base = chunk * CH
                pltpu.sync_copy(dst_hbm.at[pl.ds(base, CH)], idx_v)
                pltpu.sync_copy(ones_v, sacc.at[idx_v], add=True)

            return carry

        lax.fori_loop(0, CPT, dbody, 0)
        plsc.subcore_barrier()
        copy_out((dg0, dg1))

    return _gather_sc, _scatter_sc


BE = 1600


def _mlp_body(hu_ref, hv_ref, w1_ref, b1_ref, w2_ref, b2_ref, w3_ref, b3_ref,
              w4_ref, b4_ref, e_ref, *m_refs):
    bf = jnp.bfloat16
    hu = hu_ref[...]
    d = hu - hv_ref[...]
    dn = (((1,), (1,)), ((), ()))
    h = _leaky(lax.dot_general(d.astype(bf), w1_ref[...].astype(bf), dn,
                               preferred_element_type=jnp.float32) + b1_ref[...])
    h = _leaky(lax.dot_general(h.astype(bf), w2_ref[...].astype(bf), dn,
                               preferred_element_type=jnp.float32) + b2_ref[...])
    h = _leaky(lax.dot_general(h.astype(bf), w3_ref[...].astype(bf), dn,
                               preferred_element_type=jnp.float32) + b3_ref[...])
    z = jnp.sum(h * w4_ref[...], axis=1, keepdims=True) + b4_ref[...]
    e = 1.0 / (1.0 + jnp.exp(-z))
    e_ref[...] = e
    m = e * hu
    for i, m_ref in enumerate(m_refs):
        m_ref[...] = m[:, i * DE:(i + 1) * DE]


def _mlp_tc(hu, hv, W1, b1, W2, b2, W3, b3, W4, b4):
    full = lambda s: pl.BlockSpec(s, lambda i: (0, 0))
    espec = pl.BlockSpec((BE, DE), lambda i: (i, 0))
    return pl.pallas_call(
        _mlp_body,
        grid=(E // BE,),
        in_specs=[
            pl.BlockSpec((BE, D), lambda i: (i, 0)),
            pl.BlockSpec((BE, D), lambda i: (i, 0)),
            full((256, D)), full((1, 256)),
            full((128, 256)), full((1, 128)),
            full((64, 128)), full((1, 64)),
            full((1, 64)), full((1, 1)),
        ],
        out_specs=[pl.BlockSpec((BE, 1), lambda i: (i, 0))] + [espec] * 8,
        out_shape=[jax.ShapeDtypeStruct((E, 1), jnp.float32)]
        + [jax.ShapeDtypeStruct((E, DE), jnp.float32)] * 8,
    )(hu, hv, W1, b1.reshape(1, 256), W2, b2.reshape(1, 128),
      W3, b3.reshape(1, 64), W4, b4.reshape(1, 1))


BN = 2000


def _final_body(x_ref, s0, s1, s2, s3, s4, s5, s6, s7, dg0, dg1, ws_ref,
                w0, w1, w2, w3, w4, w5, w6, w7, b_ref, a_ref):
    deg = dg0[:, 0:1] + dg1[:, 0:1]
    inv = 1.0 / jnp.maximum(deg, 1.0)
    dn = (((1,), (1,)), ((), ()))
    rst = lax.dot_general(x_ref[...], ws_ref[...], dn,
                          preferred_element_type=jnp.float32) + b_ref[...]
    for s_ref, w_ref in zip((s0, s1, s2, s3, s4, s5, s6, s7),
                            (w0, w1, w2, w3, w4, w5, w6, w7)):
        rst = rst + lax.dot_general(s_ref[...] * inv, w_ref[...], dn,
                                    preferred_element_type=jnp.float32)
    a = _leaky(rst)
    nrm = jnp.sqrt(jnp.sum(a * a, axis=1, keepdims=True))
    a_ref[...] = a / jnp.maximum(nrm, 1e-12)


def _final_tc(x, s_parts, dg0, dg1, Wself, Wneigh, bias):
    row = pl.BlockSpec((BN, D), lambda i: (i, 0))
    erow = pl.BlockSpec((BN, DE), lambda i: (i, 0))
    full = lambda s: pl.BlockSpec(s, lambda i: (0, 0))
    wspec = full((D, DE))
    w_parts = [Wneigh[:, i * DE:(i + 1) * DE] for i in range(8)]
    return pl.pallas_call(
        _final_body,
        grid=(N // BN,),
        in_specs=[row] + [erow] * 8 + [erow, erow]
        + [full((D, D))] + [wspec] * 8 + [full((1, D))],
        out_specs=row,
        out_shape=jax.ShapeDtypeStruct((N, D), jnp.float32),
    )(x, *s_parts, dg0, dg1, Wself, *w_parts, bias.reshape(1, D))


def kernel(x, edge_index, W1, b1, W2, b2, W3, b3, W4, b4, Wself, Wneigh, bias):
    src = edge_index[0]
    dst = edge_index[1]
    _gather_sc, _scatter_sc = _sc_kernels()
    hu, hv = _gather_sc(x, src, dst)
    e, *m_parts = _mlp_tc(hu, hv, W1, b1, W2, b2, W3, b3, W4, b4)
    zs = jnp.zeros((CZ, DE), jnp.float32)
    outs = _scatter_sc(*m_parts, dst, zs)
    s_parts = [o[:N] for o in outs[:8]]
    dg0, dg1 = outs[8][:N], outs[9][:N]
    A = _final_tc(x, s_parts, dg0, dg1, Wself, Wneigh, bias)
    return (A, e)
